# baseline (speedup 1.0000x reference)
import functools

import jax
import jax.numpy as jnp
from jax import lax
from jax.experimental import pallas as pl
from jax.experimental.pallas import tpu as pltpu
from jax.experimental.pallas import tpu_sc as plsc

_LANES = 16
_CH = 64


@functools.lru_cache(maxsize=None)
def _make_sc_gather(B: int, E: int):
    info = plsc.get_sparse_core_info()
    nw = info.num_cores * info.num_subcores
    bpw = B // nw
    nch = bpw // _CH
    mesh = plsc.VectorSubcoreMesh(core_axis_name="c", subcore_axis_name="s")

    @functools.partial(
        pl.kernel,
        mesh=mesh,
        out_type=jax.ShapeDtypeStruct((B, 2 * E), jnp.float32),
        scratch_types=[
            pltpu.VMEM((bpw,), jnp.int32),
            pltpu.VMEM((bpw,), jnp.int32),
            pltpu.VMEM((bpw,), jnp.int32),
            pltpu.VMEM((bpw,), jnp.int32),
            pltpu.VMEM((_CH, 2 * E), jnp.float32),
            pltpu.VMEM((_CH, 2 * E), jnp.float32),
            pltpu.VMEM((bpw, 2 * E), jnp.float32),
            pltpu.SemaphoreType.DMA,
        ],
    )
    def sc_gather(uid_hbm, mid_hbm, utab, mtab, x_out,
                  pu, pm, au, am, bu, bm, xrows, sem):
        wid = lax.axis_index("s") * info.num_cores + lax.axis_index("c")
        base = wid * bpw
        pltpu.sync_copy(uid_hbm.at[pl.ds(base, bpw)], pu)
        pltpu.sync_copy(mid_hbm.at[pl.ds(base, bpw)], pm)

        def split(i, carry):
            sl = pl.ds(i * _LANES, _LANES)
            uv = pu[sl]
            mv = pm[sl]
            au[sl] = lax.bitwise_and(uv, 1)
            am[sl] = lax.bitwise_and(mv, 1)
            pu[sl] = lax.shift_right_logical(uv, 1)
            pm[sl] = lax.shift_right_logical(mv, 1)
            return carry

        lax.fori_loop(0, bpw // _LANES, split, 0)

        def chunk(c, carry):
            isl = pl.ds(c * _CH, _CH)
            pltpu.async_copy(utab.at[pu.at[isl]], bu, sem)
            pltpu.async_copy(mtab.at[pm.at[isl]], bm, sem)
            pltpu.make_async_copy(utab.at[pu.at[isl]], bu, sem).wait()
            pltpu.make_async_copy(mtab.at[pm.at[isl]], bm, sem).wait()
            for g in range(_CH // _LANES):
                av = au[pl.ds(c * _CH + g * _LANES, _LANES)]
                bv = am[pl.ds(c * _CH + g * _LANES, _LANES)]
                for j in range(_LANES):
                    t = g * _LANES + j
                    r = c * _CH + t
                    su = av[j]
                    sm_ = bv[j]
                    for l in range(E // _LANES):
                        lsl = pl.ds(l * _LANES, _LANES)
                        hsl = pl.ds(E + l * _LANES, _LANES)
                        xrows[r, lsl] = jnp.where(su == 1, bu[t, hsl], bu[t, lsl])
                        xrows[r, hsl] = jnp.where(sm_ == 1, bm[t, hsl], bm[t, lsl])
            return carry

        lax.fori_loop(0, nch, chunk, 0)

        pltpu.sync_copy(xrows, x_out.at[pl.ds(base, bpw)])

    return sc_gather


@functools.lru_cache(maxsize=None)
def _make_mlp(B: int, E: int, G: int, NG: int, H1: int, H2: int, bt: int):
    prec = lax.Precision.HIGHEST

    def body(x_ref, gen_ref, glen_ref, gt_ref,
             w1_ref, b1_ref, w2_ref, b2_ref, wfc_ref, bfc_ref, out_ref):
        f32 = jnp.float32
        glen = glen_ref[...]
        inv_len = 1.0 / jnp.maximum(glen, 1).astype(f32)
        iota = lax.broadcasted_iota(jnp.int32, (bt, NG), 1)
        gen = gen_ref[...]
        onehot = jnp.zeros((bt, NG), f32)
        for j in range(G):
            gj = gen[:, j:j + 1]
            wj = jnp.where(j < glen, inv_len, 0.0)
            onehot = onehot + jnp.where(gj == iota, wj, 0.0)
        gbag = jnp.dot(onehot, gt_ref[...],
                       preferred_element_type=f32, precision=prec)
        x = x_ref[...]
        u = x[:, :E]
        mr = x[:, E:] + gbag
        w1 = w1_ref[...]
        h1 = (jnp.dot(u, w1[:E, :], preferred_element_type=f32, precision=prec)
              + jnp.dot(mr, w1[E:, :], preferred_element_type=f32, precision=prec)
              + b1_ref[...])
        h1 = jnp.maximum(h1, 0.0)
        h2 = jnp.dot(h1, w2_ref[...], preferred_element_type=f32,
                     precision=prec) + b2_ref[...]
        o = jnp.dot(h2, wfc_ref[...], preferred_element_type=f32,
                    precision=prec) + bfc_ref[...]
        out_ref[...] = o

    return pl.pallas_call(
        body,
        grid=(B // bt,),
        in_specs=[
            pl.BlockSpec((bt, 2 * E), lambda i: (i, 0)),
            pl.BlockSpec((bt, G), lambda i: (i, 0)),
            pl.BlockSpec((bt, 1), lambda i: (i, 0)),
            pl.BlockSpec((NG, E), lambda i: (0, 0)),
            pl.BlockSpec((2 * E, H1), lambda i: (0, 0)),
            pl.BlockSpec((1, H1), lambda i: (0, 0)),
            pl.BlockSpec((H1, H2), lambda i: (0, 0)),
            pl.BlockSpec((1, H2), lambda i: (0, 0)),
            pl.BlockSpec((H2, 1), lambda i: (0, 0)),
            pl.BlockSpec((1, 1), lambda i: (0, 0)),
        ],
        out_specs=pl.BlockSpec((bt, 1), lambda i: (i, 0)),
        out_shape=jax.ShapeDtypeStruct((B, 1), jnp.float32),
    )


def kernel(user_data, movie_id, genres, genres_shape, user_table, movie_table,
           genre_table, W1, b1, W2, b2, Wfc, bfc):
    B = user_data.shape[0]
    E = user_table.shape[1]
    G = genres.shape[1]
    NG = genre_table.shape[0]
    H1 = W1.shape[1]
    H2 = W2.shape[1]

    one = b1[0] + 1.0
    utab2 = user_table.reshape(-1, 2 * E) * one
    mtab2 = movie_table.reshape(-1, 2 * E) * one
    x = _make_sc_gather(B, E)(user_data, movie_id, utab2, mtab2)

    mlp = _make_mlp(B, E, G, NG, H1, H2, bt=2048)
    out = mlp(x, genres, genres_shape.reshape(B, 1), genre_table,
              W1, b1.reshape(1, H1), W2, b2.reshape(1, H2),
              Wfc, bfc.reshape(1, 1))
    return out.squeeze(-1)


# speedup vs baseline: 2.2982x; 2.2982x over previous
import functools

import jax
import jax.numpy as jnp
from jax import lax
from jax.experimental import pallas as pl
from jax.experimental.pallas import tpu as pltpu
from jax.experimental.pallas import tpu_sc as plsc

_LANES = 16


@functools.lru_cache(maxsize=None)
def _make_sc_gather(B: int, E: int):
    info = plsc.get_sparse_core_info()
    nw = info.num_cores * info.num_subcores
    bpw = B // nw
    nchunks = bpw // _LANES
    mesh = plsc.VectorSubcoreMesh(core_axis_name="c", subcore_axis_name="s")

    @functools.partial(
        pl.kernel,
        mesh=mesh,
        out_type=jax.ShapeDtypeStruct((B, 2 * E), jnp.float32),
        scratch_types=[
            pltpu.VMEM((bpw,), jnp.int32),
            pltpu.VMEM((bpw,), jnp.int32),
            pltpu.VMEM((bpw, 2 * E), jnp.float32),
            pltpu.SemaphoreType.DMA,
        ],
    )
    def sc_gather(uid_hbm, mid_hbm, utab, mtab, x_out, uidx, midx, xrows, sem):
        wid = lax.axis_index("s") * info.num_cores + lax.axis_index("c")
        base = wid * bpw
        pltpu.sync_copy(uid_hbm.at[pl.ds(base, bpw)], uidx)
        pltpu.sync_copy(mid_hbm.at[pl.ds(base, bpw)], midx)

        def chunk(c, carry):
            uvec = uidx[pl.ds(c * _LANES, _LANES)]
            mvec = midx[pl.ds(c * _LANES, _LANES)]
            for j in range(_LANES):
                r = c * _LANES + j
                pltpu.async_copy(utab.at[uvec[j]], xrows.at[r, pl.ds(0, E)], sem)
                pltpu.async_copy(mtab.at[mvec[j]], xrows.at[r, pl.ds(E, E)], sem)
            @pl.when(c >= 1)
            def _():
                pltpu.make_async_copy(
                    x_out.at[pl.ds(0, _LANES)],
                    xrows.at[pl.ds(0, _LANES)], sem).wait()

            return carry

        lax.fori_loop(0, nchunks, chunk, 0)
        pltpu.make_async_copy(
            x_out.at[pl.ds(0, _LANES)], xrows.at[pl.ds(0, _LANES)], sem).wait()

        pltpu.sync_copy(xrows, x_out.at[pl.ds(base, bpw)])

    return sc_gather


@functools.lru_cache(maxsize=None)
def _make_mlp(B: int, E: int, G: int, NG: int, H1: int, H2: int, bt: int):
    prec = lax.Precision.HIGHEST

    def body(x_ref, gen_ref, glen_ref, gt_ref,
             w1_ref, b1_ref, w2_ref, b2_ref, wfc_ref, bfc_ref, out_ref):
        f32 = jnp.float32
        glen = glen_ref[...]
        inv_len = 1.0 / jnp.maximum(glen, 1).astype(f32)
        iota = lax.broadcasted_iota(jnp.int32, (bt, NG), 1)
        gen = gen_ref[...]
        onehot = jnp.zeros((bt, NG), f32)
        for j in range(G):
            gj = gen[:, j:j + 1]
            wj = jnp.where(j < glen, inv_len, 0.0)
            onehot = onehot + jnp.where(gj == iota, wj, 0.0)
        gbag = jnp.dot(onehot, gt_ref[...],
                       preferred_element_type=f32, precision=prec)
        x = x_ref[...]
        u = x[:, :E]
        mr = x[:, E:] + gbag
        w1 = w1_ref[...]
        h1 = (jnp.dot(u, w1[:E, :], preferred_element_type=f32, precision=prec)
              + jnp.dot(mr, w1[E:, :], preferred_element_type=f32, precision=prec)
              + b1_ref[...])
        h1 = jnp.maximum(h1, 0.0)
        h2 = jnp.dot(h1, w2_ref[...], preferred_element_type=f32,
                     precision=prec) + b2_ref[...]
        o = jnp.dot(h2, wfc_ref[...], preferred_element_type=f32,
                    precision=prec) + bfc_ref[...]
        out_ref[...] = o

    return pl.pallas_call(
        body,
        grid=(B // bt,),
        in_specs=[
            pl.BlockSpec((bt, 2 * E), lambda i: (i, 0)),
            pl.BlockSpec((bt, G), lambda i: (i, 0)),
            pl.BlockSpec((bt, 1), lambda i: (i, 0)),
            pl.BlockSpec((NG, E), lambda i: (0, 0)),
            pl.BlockSpec((2 * E, H1), lambda i: (0, 0)),
            pl.BlockSpec((1, H1), lambda i: (0, 0)),
            pl.BlockSpec((H1, H2), lambda i: (0, 0)),
            pl.BlockSpec((1, H2), lambda i: (0, 0)),
            pl.BlockSpec((H2, 1), lambda i: (0, 0)),
            pl.BlockSpec((1, 1), lambda i: (0, 0)),
        ],
        out_specs=pl.BlockSpec((bt, 1), lambda i: (i, 0)),
        out_shape=jax.ShapeDtypeStruct((B, 1), jnp.float32),
    )


def kernel(user_data, movie_id, genres, genres_shape, user_table, movie_table,
           genre_table, W1, b1, W2, b2, Wfc, bfc):
    B = user_data.shape[0]
    E = user_table.shape[1]
    G = genres.shape[1]
    NG = genre_table.shape[0]
    H1 = W1.shape[1]
    H2 = W2.shape[1]

    x = _make_sc_gather(B, E)(user_data, movie_id, user_table, movie_table)

    mlp = _make_mlp(B, E, G, NG, H1, H2, bt=2048)
    out = mlp(x, genres, genres_shape.reshape(B, 1), genre_table,
              W1, b1.reshape(1, H1), W2, b2.reshape(1, H2),
              Wfc, bfc.reshape(1, 1))
    return out.squeeze(-1)
